# baseline (speedup 1.0000x reference)
import functools

import jax
import jax.numpy as jnp
from jax import lax
from jax.experimental import pallas as pl
from jax.experimental.pallas import tpu as pltpu
from jax.experimental.pallas import tpu_sc as plsc

B, S, DIM = 1, 2048, 768
E, TOP_K = 8, 2
HID = 4 * DIM
TM = 256
PMAX = S * TOP_K + E * TM
NT = PMAX // TM
NTC = 64
NW = 32
TPW = S // NW
CHUNK = 256
WV = 128


NCH = S // CHUNK


def _gate_body(temp_ref, x_ref, Wg_ref, bg_ref,
               d0_ref, d1_ref, w0_ref, w1_ref, te_ref, tv_ref,
               i1s, i2s, c0s, c1s, carry_s):
    i = pl.program_id(0)

    @pl.when(i == 0)
    def _init():
        carry_s[...] = jnp.zeros((1, E), jnp.float32)

    @pl.when(i < NCH)
    def _gate_chunk():
        iota_e = lax.broadcasted_iota(jnp.int32, (CHUNK, E), 1)
        g = jnp.dot(x_ref[...], Wg_ref[...],
                    preferred_element_type=jnp.float32)
        g = (g + bg_ref[...]) / temp_ref[...]
        m1 = jnp.max(g, axis=1, keepdims=True)
        i1 = jnp.min(jnp.where(g == m1, iota_e, E), axis=1, keepdims=True)
        g2 = jnp.where(iota_e == i1, -jnp.inf, g)
        m2 = jnp.max(g2, axis=1, keepdims=True)
        i2 = jnp.min(jnp.where(g2 == m2, iota_e, E), axis=1, keepdims=True)
        p = jnp.exp(m2 - m1)
        w0_ref[...] = jnp.broadcast_to(1.0 / (1.0 + p), (CHUNK, WV))
        w1_ref[...] = jnp.broadcast_to(p / (1.0 + p), (CHUNK, WV))
        oh1 = jnp.where(iota_e == i1, 1.0, 0.0)
        oh2 = jnp.where(iota_e == i2, 1.0, 0.0)
        sall = oh1 + oh2
        li = lax.broadcasted_iota(jnp.int32, (CHUNK, CHUNK), 0)
        lj = lax.broadcasted_iota(jnp.int32, (CHUNK, CHUNK), 1)
        ltri = jnp.where(li > lj, 1.0, 0.0)
        cexcl = (jnp.dot(ltri, sall, preferred_element_type=jnp.float32)
                 + carry_s[...])
        carry_s[...] += jnp.sum(sall, axis=0, keepdims=True)
        sl = pl.ds(i * CHUNK, CHUNK)
        i1s[sl, :] = i1
        i2s[sl, :] = i2
        c0s[sl, :] = jnp.sum(cexcl * oh1, axis=1, keepdims=True)
        c1s[sl, :] = jnp.sum(cexcl * oh2, axis=1, keepdims=True)

    @pl.when(i == NCH)
    def _finalize():
        carry = carry_s[...]
        cnt_pad = jnp.ceil(carry / TM) * TM
        ui = lax.broadcasted_iota(jnp.int32, (E, E), 0)
        uj = lax.broadcasted_iota(jnp.int32, (E, E), 1)
        utri = jnp.where(ui < uj, 1.0, 0.0)
        offs = jnp.dot(cnt_pad, utri,
                       preferred_element_type=jnp.float32)

        iota_s = lax.broadcasted_iota(jnp.int32, (S, E), 1)
        offs_s = jnp.broadcast_to(offs, (S, E))
        o0 = jnp.sum(jnp.where(iota_s == i1s[...], offs_s, 0.0),
                     axis=1, keepdims=True)
        o1 = jnp.sum(jnp.where(iota_s == i2s[...], offs_s, 0.0),
                     axis=1, keepdims=True)
        d0_ref[...] = (c0s[...] + o0).astype(jnp.int32)
        d1_ref[...] = (c1s[...] + o1).astype(jnp.int32)

        tbase = (lax.broadcasted_iota(jnp.int32, (NTC, E), 0)
                 .astype(jnp.float32) * TM)
        offs_b = jnp.broadcast_to(offs, (NTC, E))
        cpad_b = jnp.broadcast_to(cnt_pad, (NTC, E))
        ind = jnp.where((tbase >= offs_b) & (tbase < offs_b + cpad_b),
                        1.0, 0.0)
        eids = lax.broadcasted_iota(jnp.int32, (NTC, E), 1).astype(jnp.float32)
        texp = jnp.sum(ind * eids, axis=1, keepdims=True)
        tval = jnp.sum(ind, axis=1, keepdims=True)
        te_ref[...] = jnp.where(tval > 0, texp,
                                float(E - 1)).astype(jnp.int32)[:NT]
        tv_ref[...] = tval.astype(jnp.int32)[:NT]


def _gate_call(x2, Wg, bg2, temp):
    return pl.pallas_call(
        _gate_body,
        grid=(NCH + 1,),
        in_specs=[
            pl.BlockSpec((1, 1), lambda i: (0, 0)),
            pl.BlockSpec((CHUNK, DIM), lambda i: (jnp.minimum(i, NCH - 1), 0)),
            pl.BlockSpec((DIM, E), lambda i: (0, 0)),
            pl.BlockSpec((1, E), lambda i: (0, 0)),
        ],
        out_specs=[
            pl.BlockSpec((S, 1), lambda i: (0, 0)),
            pl.BlockSpec((S, 1), lambda i: (0, 0)),
            pl.BlockSpec((CHUNK, WV), lambda i: (jnp.minimum(i, NCH - 1), 0)),
            pl.BlockSpec((CHUNK, WV), lambda i: (jnp.minimum(i, NCH - 1), 0)),
            pl.BlockSpec((NT, 1), lambda i: (0, 0)),
            pl.BlockSpec((NT, 1), lambda i: (0, 0)),
        ],
        out_shape=[
            jax.ShapeDtypeStruct((S, 1), jnp.int32),
            jax.ShapeDtypeStruct((S, 1), jnp.int32),
            jax.ShapeDtypeStruct((S, WV), jnp.float32),
            jax.ShapeDtypeStruct((S, WV), jnp.float32),
            jax.ShapeDtypeStruct((NT, 1), jnp.int32),
            jax.ShapeDtypeStruct((NT, 1), jnp.int32),
        ],
        scratch_shapes=[
            pltpu.VMEM((S, 1), jnp.int32),
            pltpu.VMEM((S, 1), jnp.int32),
            pltpu.VMEM((S, 1), jnp.float32),
            pltpu.VMEM((S, 1), jnp.float32),
            pltpu.VMEM((1, E), jnp.float32),
        ],
        compiler_params=pltpu.CompilerParams(
            dimension_semantics=("arbitrary",),
        ),
    )(temp, x2, Wg, bg2)


def _dispatch_call(x2, d0, d1, w0w, w1w):
    mesh = plsc.VectorSubcoreMesh(core_axis_name="c", subcore_axis_name="s")

    @functools.partial(
        pl.kernel, mesh=mesh,
        out_type=[
            jax.ShapeDtypeStruct((PMAX, DIM), jnp.float32),
            jax.ShapeDtypeStruct((PMAX, WV), jnp.float32),
        ],
        scratch_types=[
            pltpu.VMEM((TPW, DIM), jnp.float32),
            pltpu.VMEM((TPW,), jnp.int32),
            pltpu.VMEM((TPW,), jnp.int32),
            pltpu.VMEM((TPW, WV), jnp.float32),
            pltpu.VMEM((TPW, WV), jnp.float32),
            pltpu.SemaphoreType.DMA,
        ],
    )
    def k2(x_hbm, d0_hbm, d1_hbm, w0_hbm, w1_hbm, xs_hbm, ws_hbm,
           xbuf, i0, i1, wb0, wb1, sem):
        wid = lax.axis_index("s") * 2 + lax.axis_index("c")
        base = wid * TPW
        a0 = pltpu.async_copy(x_hbm.at[pl.ds(base, TPW)], xbuf, sem)
        a1 = pltpu.async_copy(d0_hbm.at[pl.ds(base, TPW)], i0, sem)
        a2 = pltpu.async_copy(d1_hbm.at[pl.ds(base, TPW)], i1, sem)
        a3 = pltpu.async_copy(w0_hbm.at[pl.ds(base, TPW)], wb0, sem)
        a4 = pltpu.async_copy(w1_hbm.at[pl.ds(base, TPW)], wb1, sem)
        a0.wait()
        a1.wait()
        a2.wait()
        a3.wait()
        a4.wait()
        c0 = pltpu.async_copy(xbuf, xs_hbm.at[i0], sem)
        c1 = pltpu.async_copy(xbuf, xs_hbm.at[i1], sem)
        c2 = pltpu.async_copy(wb0, ws_hbm.at[i0], sem)
        c3 = pltpu.async_copy(wb1, ws_hbm.at[i1], sem)
        c0.wait()
        c1.wait()
        c2.wait()
        c3.wait()

    return k2(x2, d0, d1, w0w, w1w)


def _ffn_body(texp_ref, tval_ref, xs_ref, W1_ref, b1_ref, W2_ref, b2_ref,
              ws_ref, y_ref):
    i = pl.program_id(0)

    @pl.when(tval_ref[i] > 0)
    def _():
        e = texp_ref[i]
        xt = xs_ref[...]
        h = jnp.dot(xt, W1_ref[0], preferred_element_type=jnp.float32)
        h = h + b1_ref[pl.ds(e, 1), :]
        h = 0.5 * h * (1.0 + lax.erf(h * 0.7071067811865476))
        y = jnp.dot(h, W2_ref[0], preferred_element_type=jnp.float32)
        y = y + b2_ref[pl.ds(e, 1), :]
        y_ref[...] = y * ws_ref[:, :1]


def _ffn_call(xs, ws, W1, b1, W2, b2, texp, tval):
    grid_spec = pltpu.PrefetchScalarGridSpec(
        num_scalar_prefetch=2,
        grid=(NT,),
        in_specs=[
            pl.BlockSpec((TM, DIM), lambda i, te, tv: (i, 0)),
            pl.BlockSpec((1, DIM, HID), lambda i, te, tv: (te[i], 0, 0)),
            pl.BlockSpec((E, HID), lambda i, te, tv: (0, 0)),
            pl.BlockSpec((1, HID, DIM), lambda i, te, tv: (te[i], 0, 0)),
            pl.BlockSpec((E, DIM), lambda i, te, tv: (0, 0)),
            pl.BlockSpec((TM, WV), lambda i, te, tv: (i, 0)),
        ],
        out_specs=pl.BlockSpec((TM, DIM), lambda i, te, tv: (i, 0)),
    )
    return pl.pallas_call(
        _ffn_body,
        grid_spec=grid_spec,
        out_shape=jax.ShapeDtypeStruct((PMAX, DIM), jnp.float32),
        compiler_params=pltpu.CompilerParams(
            dimension_semantics=("arbitrary",),
        ),
    )(texp, tval, xs, W1, b1, W2, b2, ws)


def _combine_call(ys, d0, d1):
    mesh = plsc.VectorSubcoreMesh(core_axis_name="c", subcore_axis_name="s")
    half = TPW // 2

    @functools.partial(
        pl.kernel, mesh=mesh,
        out_type=jax.ShapeDtypeStruct((B, S, DIM), jnp.float32),
        scratch_types=[
            pltpu.VMEM((half,), jnp.int32),
            pltpu.VMEM((half,), jnp.int32),
            pltpu.VMEM((half,), jnp.int32),
            pltpu.VMEM((half,), jnp.int32),
            pltpu.VMEM((half, DIM), jnp.float32),
            pltpu.VMEM((half, DIM), jnp.float32),
            pltpu.VMEM((half, DIM), jnp.float32),
            pltpu.VMEM((half, DIM), jnp.float32),
            pltpu.SemaphoreType.DMA,
            pltpu.SemaphoreType.DMA,
            pltpu.SemaphoreType.DMA,
        ],
    )
    def k4(y_hbm, d0_hbm, d1_hbm, out_hbm,
           i0a, i1a, i0b, i1b, ya0, ya1, yb0, yb1, sema, semb, sems):
        wid = lax.axis_index("s") * 2 + lax.axis_index("c")
        base = wid * TPW
        pltpu.sync_copy(d0_hbm.at[pl.ds(base, half)], i0a)
        pltpu.sync_copy(d1_hbm.at[pl.ds(base, half)], i1a)
        ga0 = pltpu.async_copy(y_hbm.at[i0a], ya0, sema)
        ga1 = pltpu.async_copy(y_hbm.at[i1a], ya1, sema)
        pltpu.sync_copy(d0_hbm.at[pl.ds(base + half, half)], i0b)
        pltpu.sync_copy(d1_hbm.at[pl.ds(base + half, half)], i1b)
        gb0 = pltpu.async_copy(y_hbm.at[i0b], yb0, semb)
        gb1 = pltpu.async_copy(y_hbm.at[i1b], yb1, semb)

        def addrows(p0, p1):
            def row(r, carry):
                for cc in range(DIM // 16):
                    sl = pl.ds(cc * 16, 16)
                    p0[r, sl] = p0[r, sl] + p1[r, sl]
                return carry
            lax.fori_loop(0, half, row, 0)

        ga0.wait()
        ga1.wait()
        addrows(ya0, ya1)
        sa = pltpu.async_copy(ya0, out_hbm.at[0, pl.ds(base, half)], sems)
        gb0.wait()
        gb1.wait()
        addrows(yb0, yb1)
        sb = pltpu.async_copy(yb0, out_hbm.at[0, pl.ds(base + half, half)], sems)
        sa.wait()
        sb.wait()

    return k4(ys, d0, d1)


def kernel(x, W1, b1, W2, b2, Wg, bg, temperature):
    x2 = x.reshape(S, DIM)
    bg2 = bg.reshape(1, E)
    temp = jnp.reshape(temperature, (1, 1)).astype(jnp.float32)

    d0c, d1c, w0w, w1w, te, tv = _gate_call(x2, Wg, bg2, temp)
    d0 = d0c.reshape(S)
    d1 = d1c.reshape(S)
    xs, ws = _dispatch_call(x2, d0, d1, w0w, w1w)
    ys = _ffn_call(xs, ws, W1, b1, W2, b2, te.reshape(NT), tv.reshape(NT))
    return _combine_call(ys, d0, d1)


# speedup vs baseline: 1.1872x; 1.1872x over previous
import functools

import jax
import jax.numpy as jnp
from jax import lax
from jax.experimental import pallas as pl
from jax.experimental.pallas import tpu as pltpu
from jax.experimental.pallas import tpu_sc as plsc

B, S, DIM = 1, 2048, 768
E, TOP_K = 8, 2
HID = 4 * DIM
TM = 256
PMAX = S * TOP_K + E * TM
NT = PMAX // TM
NTC = 64
NW = 32
TPW = S // NW
CHUNK = 256
WV = 128


def _gate_body(temp_ref, x_ref, Wg_ref, bg_ref,
               d0_ref, d1_ref, w0_ref, w1_ref, te_ref, tv_ref):
    iota_e = lax.broadcasted_iota(jnp.int32, (S, E), 1)
    g = jnp.dot(x_ref[...], Wg_ref[...], preferred_element_type=jnp.float32)
    g = (g + bg_ref[...]) / temp_ref[...]
    m1 = jnp.max(g, axis=1, keepdims=True)
    i1 = jnp.min(jnp.where(g == m1, iota_e, E), axis=1, keepdims=True)
    g2 = jnp.where(iota_e == i1, -jnp.inf, g)
    m2 = jnp.max(g2, axis=1, keepdims=True)
    i2 = jnp.min(jnp.where(g2 == m2, iota_e, E), axis=1, keepdims=True)
    p = jnp.exp(m2 - m1)
    wa = 1.0 / (1.0 + p)
    wb = p / (1.0 + p)
    oh1 = jnp.where(iota_e == i1, 1.0, 0.0)
    oh2 = jnp.where(iota_e == i2, 1.0, 0.0)
    sall = oh1 + oh2

    li = lax.broadcasted_iota(jnp.int32, (CHUNK, CHUNK), 0)
    lj = lax.broadcasted_iota(jnp.int32, (CHUNK, CHUNK), 1)
    ltri = jnp.where(li > lj, 1.0, 0.0)
    carry = jnp.zeros((1, E), jnp.float32)
    chunks = []
    for c in range(S // CHUNK):
        sc_ = sall[c * CHUNK:(c + 1) * CHUNK]
        chunks.append(
            jnp.dot(ltri, sc_, preferred_element_type=jnp.float32) + carry)
        carry = carry + jnp.sum(sc_, axis=0, keepdims=True)
    cexcl = jnp.concatenate(chunks, axis=0) if len(chunks) > 1 else chunks[0]

    cnt_pad = jnp.ceil(carry / TM) * TM
    ui = lax.broadcasted_iota(jnp.int32, (E, E), 0)
    uj = lax.broadcasted_iota(jnp.int32, (E, E), 1)
    utri = jnp.where(ui < uj, 1.0, 0.0)
    offs = jnp.dot(cnt_pad, utri, preferred_element_type=jnp.float32)

    pos = cexcl + offs
    d0 = jnp.sum(pos * oh1, axis=1, keepdims=True)
    d1 = jnp.sum(pos * oh2, axis=1, keepdims=True)
    d0_ref[...] = d0.astype(jnp.int32)
    d1_ref[...] = d1.astype(jnp.int32)
    w0_ref[...] = jnp.broadcast_to(wa, (S, WV))
    w1_ref[...] = jnp.broadcast_to(wb, (S, WV))

    tbase = lax.broadcasted_iota(jnp.int32, (NTC, E), 0).astype(jnp.float32) * TM
    offs_b = jnp.broadcast_to(offs, (NTC, E))
    cpad_b = jnp.broadcast_to(cnt_pad, (NTC, E))
    ind = jnp.where((tbase >= offs_b) & (tbase < offs_b + cpad_b), 1.0, 0.0)
    eids = lax.broadcasted_iota(jnp.int32, (NTC, E), 1).astype(jnp.float32)
    texp = jnp.sum(ind * eids, axis=1, keepdims=True)
    tval = jnp.sum(ind, axis=1, keepdims=True)
    te_ref[...] = jnp.where(tval > 0, texp, float(E - 1)).astype(jnp.int32)[:NT]
    tv_ref[...] = tval.astype(jnp.int32)[:NT]


def _gate_call(x2, Wg, bg2, temp):
    return pl.pallas_call(
        _gate_body,
        grid=(1,),
        in_specs=[
            pl.BlockSpec((1, 1), lambda i: (0, 0)),
            pl.BlockSpec((S, DIM), lambda i: (0, 0)),
            pl.BlockSpec((DIM, E), lambda i: (0, 0)),
            pl.BlockSpec((1, E), lambda i: (0, 0)),
        ],
        out_specs=[
            pl.BlockSpec((S, 1), lambda i: (0, 0)),
            pl.BlockSpec((S, 1), lambda i: (0, 0)),
            pl.BlockSpec((S, WV), lambda i: (0, 0)),
            pl.BlockSpec((S, WV), lambda i: (0, 0)),
            pl.BlockSpec((NT, 1), lambda i: (0, 0)),
            pl.BlockSpec((NT, 1), lambda i: (0, 0)),
        ],
        out_shape=[
            jax.ShapeDtypeStruct((S, 1), jnp.int32),
            jax.ShapeDtypeStruct((S, 1), jnp.int32),
            jax.ShapeDtypeStruct((S, WV), jnp.float32),
            jax.ShapeDtypeStruct((S, WV), jnp.float32),
            jax.ShapeDtypeStruct((NT, 1), jnp.int32),
            jax.ShapeDtypeStruct((NT, 1), jnp.int32),
        ],
    )(temp, x2, Wg, bg2)


def _dispatch_call(x2, d0, d1, w0w, w1w):
    mesh = plsc.VectorSubcoreMesh(core_axis_name="c", subcore_axis_name="s")

    @functools.partial(
        pl.kernel, mesh=mesh,
        out_type=[
            jax.ShapeDtypeStruct((PMAX, DIM), jnp.float32),
            jax.ShapeDtypeStruct((PMAX, WV), jnp.float32),
        ],
        scratch_types=[
            pltpu.VMEM((TPW, DIM), jnp.float32),
            pltpu.VMEM((TPW,), jnp.int32),
            pltpu.VMEM((TPW,), jnp.int32),
            pltpu.VMEM((TPW, WV), jnp.float32),
            pltpu.VMEM((TPW, WV), jnp.float32),
            pltpu.SemaphoreType.DMA,
        ],
    )
    def k2(x_hbm, d0_hbm, d1_hbm, w0_hbm, w1_hbm, xs_hbm, ws_hbm,
           xbuf, i0, i1, wb0, wb1, sem):
        wid = lax.axis_index("s") * 2 + lax.axis_index("c")
        base = wid * TPW
        a0 = pltpu.async_copy(x_hbm.at[pl.ds(base, TPW)], xbuf, sem)
        a1 = pltpu.async_copy(d0_hbm.at[pl.ds(base, TPW)], i0, sem)
        a2 = pltpu.async_copy(d1_hbm.at[pl.ds(base, TPW)], i1, sem)
        a3 = pltpu.async_copy(w0_hbm.at[pl.ds(base, TPW)], wb0, sem)
        a4 = pltpu.async_copy(w1_hbm.at[pl.ds(base, TPW)], wb1, sem)
        a0.wait()
        a1.wait()
        a2.wait()
        a3.wait()
        a4.wait()
        c0 = pltpu.async_copy(xbuf, xs_hbm.at[i0], sem)
        c1 = pltpu.async_copy(xbuf, xs_hbm.at[i1], sem)
        c2 = pltpu.async_copy(wb0, ws_hbm.at[i0], sem)
        c3 = pltpu.async_copy(wb1, ws_hbm.at[i1], sem)
        c0.wait()
        c1.wait()
        c2.wait()
        c3.wait()

    return k2(x2, d0, d1, w0w, w1w)


def _ffn_body(texp_ref, tval_ref, gfirst_ref, gslot_ref, gnxt_ref, ghn_ref,
              xs_ref, W1_ref, b1_ref, W2_ref, b2_ref, ws_ref, y_ref,
              w1s, w2s, sem1, sem2):
    i = pl.program_id(0)
    slot = gslot_ref[i]

    def issue(eidx, sl):
        pltpu.make_async_copy(W1_ref.at[eidx], w1s.at[sl], sem1.at[sl]).start()
        pltpu.make_async_copy(W2_ref.at[eidx], w2s.at[sl], sem2.at[sl]).start()

    def drain(eidx, sl):
        pltpu.make_async_copy(W1_ref.at[eidx], w1s.at[sl], sem1.at[sl]).wait()
        pltpu.make_async_copy(W2_ref.at[eidx], w2s.at[sl], sem2.at[sl]).wait()

    @pl.when(i == 0)
    def _prime():
        issue(texp_ref[0], slot)

    @pl.when(gfirst_ref[i] == 1)
    def _group_start():
        drain(texp_ref[i], slot)

        @pl.when(ghn_ref[i] == 1)
        def _():
            issue(gnxt_ref[i], 1 - slot)

    @pl.when(tval_ref[i] > 0)
    def _():
        e = texp_ref[i]
        xt = xs_ref[...]
        h = jnp.dot(xt, w1s[slot], preferred_element_type=jnp.float32)
        h = h + b1_ref[pl.ds(e, 1), :]
        h = 0.5 * h * (1.0 + lax.erf(h * 0.7071067811865476))
        y = jnp.dot(h, w2s[slot], preferred_element_type=jnp.float32)
        y = y + b2_ref[pl.ds(e, 1), :]
        y_ref[...] = y * ws_ref[:, :1]


def _ffn_call(xs, ws, W1, b1, W2, b2, texp, tval, gfirst, gslot, gnxt, ghn):
    grid_spec = pltpu.PrefetchScalarGridSpec(
        num_scalar_prefetch=6,
        grid=(NT,),
        in_specs=[
            pl.BlockSpec((TM, DIM), lambda i, *_: (i, 0)),
            pl.BlockSpec(memory_space=pl.ANY),
            pl.BlockSpec((E, HID), lambda i, *_: (0, 0)),
            pl.BlockSpec(memory_space=pl.ANY),
            pl.BlockSpec((E, DIM), lambda i, *_: (0, 0)),
            pl.BlockSpec((TM, WV), lambda i, *_: (i, 0)),
        ],
        out_specs=pl.BlockSpec((TM, DIM), lambda i, *_: (i, 0)),
        scratch_shapes=[
            pltpu.VMEM((2, DIM, HID), jnp.float32),
            pltpu.VMEM((2, HID, DIM), jnp.float32),
            pltpu.SemaphoreType.DMA((2,)),
            pltpu.SemaphoreType.DMA((2,)),
        ],
    )
    return pl.pallas_call(
        _ffn_body,
        grid_spec=grid_spec,
        out_shape=jax.ShapeDtypeStruct((PMAX, DIM), jnp.float32),
        compiler_params=pltpu.CompilerParams(
            dimension_semantics=("arbitrary",),
        ),
    )(texp, tval, gfirst, gslot, gnxt, ghn, xs, W1, b1, W2, b2, ws)


def _group_meta(te):
    idx = jnp.arange(NT, dtype=jnp.int32)
    first = jnp.concatenate(
        [jnp.ones((1,), jnp.int32),
         (te[1:] != te[:-1]).astype(jnp.int32)])
    gslot = (jnp.cumsum(first) - 1) % 2
    sstart = jnp.where(first == 1, idx, NT + 1)
    suf = jnp.flip(jax.lax.cummin(jnp.flip(sstart)))
    nxtpos = jnp.concatenate([suf[1:], jnp.full((1,), NT + 1, jnp.int32)])
    ghn = (nxtpos <= NT - 1).astype(jnp.int32)
    gnxt = te[jnp.clip(nxtpos, 0, NT - 1)]
    return first, gslot.astype(jnp.int32), gnxt, ghn


def _combine_call(ys, d0, d1):
    mesh = plsc.VectorSubcoreMesh(core_axis_name="c", subcore_axis_name="s")
    half = TPW // 2

    @functools.partial(
        pl.kernel, mesh=mesh,
        out_type=jax.ShapeDtypeStruct((B, S, DIM), jnp.float32),
        scratch_types=[
            pltpu.VMEM((half,), jnp.int32),
            pltpu.VMEM((half,), jnp.int32),
            pltpu.VMEM((half,), jnp.int32),
            pltpu.VMEM((half,), jnp.int32),
            pltpu.VMEM((half, DIM), jnp.float32),
            pltpu.VMEM((half, DIM), jnp.float32),
            pltpu.VMEM((half, DIM), jnp.float32),
            pltpu.VMEM((half, DIM), jnp.float32),
            pltpu.SemaphoreType.DMA,
            pltpu.SemaphoreType.DMA,
            pltpu.SemaphoreType.DMA,
        ],
    )
    def k4(y_hbm, d0_hbm, d1_hbm, out_hbm,
           i0a, i1a, i0b, i1b, ya0, ya1, yb0, yb1, sema, semb, sems):
        wid = lax.axis_index("s") * 2 + lax.axis_index("c")
        base = wid * TPW
        pltpu.sync_copy(d0_hbm.at[pl.ds(base, half)], i0a)
        pltpu.sync_copy(d1_hbm.at[pl.ds(base, half)], i1a)
        ga0 = pltpu.async_copy(y_hbm.at[i0a], ya0, sema)
        ga1 = pltpu.async_copy(y_hbm.at[i1a], ya1, sema)
        pltpu.sync_copy(d0_hbm.at[pl.ds(base + half, half)], i0b)
        pltpu.sync_copy(d1_hbm.at[pl.ds(base + half, half)], i1b)
        gb0 = pltpu.async_copy(y_hbm.at[i0b], yb0, semb)
        gb1 = pltpu.async_copy(y_hbm.at[i1b], yb1, semb)

        def addrows(p0, p1):
            def row(r, carry):
                for cc in range(DIM // 16):
                    sl = pl.ds(cc * 16, 16)
                    p0[r, sl] = p0[r, sl] + p1[r, sl]
                return carry
            lax.fori_loop(0, half, row, 0)

        ga0.wait()
        ga1.wait()
        addrows(ya0, ya1)
        sa = pltpu.async_copy(ya0, out_hbm.at[0, pl.ds(base, half)], sems)
        gb0.wait()
        gb1.wait()
        addrows(yb0, yb1)
        sb = pltpu.async_copy(yb0, out_hbm.at[0, pl.ds(base + half, half)], sems)
        sa.wait()
        sb.wait()

    return k4(ys, d0, d1)


def kernel(x, W1, b1, W2, b2, Wg, bg, temperature):
    x2 = x.reshape(S, DIM)
    bg2 = bg.reshape(1, E)
    temp = jnp.reshape(temperature, (1, 1)).astype(jnp.float32)

    d0c, d1c, w0w, w1w, te, tv = _gate_call(x2, Wg, bg2, temp)
    d0 = d0c.reshape(S)
    d1 = d1c.reshape(S)
    xs, ws = _dispatch_call(x2, d0, d1, w0w, w1w)
    ter = te.reshape(NT)
    gfirst, gslot, gnxt, ghn = _group_meta(ter)
    ys = _ffn_call(xs, ws, W1, b1, W2, b2, ter, tv.reshape(NT),
                   gfirst, gslot, gnxt, ghn)
    return _combine_call(ys, d0, d1)


# speedup vs baseline: 1.1949x; 1.0065x over previous
import functools

import jax
import jax.numpy as jnp
from jax import lax
from jax.experimental import pallas as pl
from jax.experimental.pallas import tpu as pltpu
from jax.experimental.pallas import tpu_sc as plsc

B, S, DIM = 1, 2048, 768
E, TOP_K = 8, 2
HID = 4 * DIM
TM = 256
PMAX = S * TOP_K + E * TM
NT = PMAX // TM
NTC = 64
NW = 32
TPW = S // NW
CHUNK = 256
WV = 128


def _gate_body(temp_ref, x_ref, Wg_ref, bg_ref,
               d0_ref, d1_ref, w0_ref, w1_ref, te_ref, tv_ref):
    iota_e = lax.broadcasted_iota(jnp.int32, (S, E), 1)
    g = jnp.dot(x_ref[...], Wg_ref[...], preferred_element_type=jnp.float32)
    g = (g + bg_ref[...]) / temp_ref[...]
    m1 = jnp.max(g, axis=1, keepdims=True)
    i1 = jnp.min(jnp.where(g == m1, iota_e, E), axis=1, keepdims=True)
    g2 = jnp.where(iota_e == i1, -jnp.inf, g)
    m2 = jnp.max(g2, axis=1, keepdims=True)
    i2 = jnp.min(jnp.where(g2 == m2, iota_e, E), axis=1, keepdims=True)
    p = jnp.exp(m2 - m1)
    wa = 1.0 / (1.0 + p)
    wb = p / (1.0 + p)
    oh1 = jnp.where(iota_e == i1, 1.0, 0.0)
    oh2 = jnp.where(iota_e == i2, 1.0, 0.0)
    sall = oh1 + oh2

    li = lax.broadcasted_iota(jnp.int32, (CHUNK, CHUNK), 0)
    lj = lax.broadcasted_iota(jnp.int32, (CHUNK, CHUNK), 1)
    ltri = jnp.where(li > lj, 1.0, 0.0)
    carry = jnp.zeros((1, E), jnp.float32)
    chunks = []
    for c in range(S // CHUNK):
        sc_ = sall[c * CHUNK:(c + 1) * CHUNK]
        chunks.append(
            jnp.dot(ltri, sc_, preferred_element_type=jnp.float32) + carry)
        carry = carry + jnp.sum(sc_, axis=0, keepdims=True)
    cexcl = jnp.concatenate(chunks, axis=0) if len(chunks) > 1 else chunks[0]

    cnt_pad = jnp.ceil(carry / TM) * TM
    ui = lax.broadcasted_iota(jnp.int32, (E, E), 0)
    uj = lax.broadcasted_iota(jnp.int32, (E, E), 1)
    utri = jnp.where(ui < uj, 1.0, 0.0)
    offs = jnp.dot(cnt_pad, utri, preferred_element_type=jnp.float32)

    pos = cexcl + offs
    d0 = jnp.sum(pos * oh1, axis=1, keepdims=True)
    d1 = jnp.sum(pos * oh2, axis=1, keepdims=True)
    d0_ref[...] = d0.astype(jnp.int32)
    d1_ref[...] = d1.astype(jnp.int32)
    w0_ref[...] = jnp.broadcast_to(wa, (S, WV))
    w1_ref[...] = jnp.broadcast_to(wb, (S, WV))

    tbase = lax.broadcasted_iota(jnp.int32, (NTC, E), 0).astype(jnp.float32) * TM
    offs_b = jnp.broadcast_to(offs, (NTC, E))
    cpad_b = jnp.broadcast_to(cnt_pad, (NTC, E))
    ind = jnp.where((tbase >= offs_b) & (tbase < offs_b + cpad_b), 1.0, 0.0)
    eids = lax.broadcasted_iota(jnp.int32, (NTC, E), 1).astype(jnp.float32)
    texp = jnp.sum(ind * eids, axis=1, keepdims=True)
    tval = jnp.sum(ind, axis=1, keepdims=True)
    te_ref[...] = jnp.where(tval > 0, texp, float(E - 1)).astype(jnp.int32)[:NT]
    tv_ref[...] = tval.astype(jnp.int32)[:NT]


def _gate_call(x2, Wg, bg2, temp):
    return pl.pallas_call(
        _gate_body,
        grid=(1,),
        in_specs=[
            pl.BlockSpec((1, 1), lambda i: (0, 0)),
            pl.BlockSpec((S, DIM), lambda i: (0, 0)),
            pl.BlockSpec((DIM, E), lambda i: (0, 0)),
            pl.BlockSpec((1, E), lambda i: (0, 0)),
        ],
        out_specs=[
            pl.BlockSpec((S, 1), lambda i: (0, 0)),
            pl.BlockSpec((S, 1), lambda i: (0, 0)),
            pl.BlockSpec((S, WV), lambda i: (0, 0)),
            pl.BlockSpec((S, WV), lambda i: (0, 0)),
            pl.BlockSpec((NT, 1), lambda i: (0, 0)),
            pl.BlockSpec((NT, 1), lambda i: (0, 0)),
        ],
        out_shape=[
            jax.ShapeDtypeStruct((S, 1), jnp.int32),
            jax.ShapeDtypeStruct((S, 1), jnp.int32),
            jax.ShapeDtypeStruct((S, WV), jnp.float32),
            jax.ShapeDtypeStruct((S, WV), jnp.float32),
            jax.ShapeDtypeStruct((NT, 1), jnp.int32),
            jax.ShapeDtypeStruct((NT, 1), jnp.int32),
        ],
    )(temp, x2, Wg, bg2)


def _dispatch_call(x2, d0, d1, w0w, w1w):
    mesh = plsc.VectorSubcoreMesh(core_axis_name="c", subcore_axis_name="s")

    @functools.partial(
        pl.kernel, mesh=mesh,
        out_type=[
            jax.ShapeDtypeStruct((PMAX, DIM), jnp.float32),
            jax.ShapeDtypeStruct((PMAX, WV), jnp.float32),
        ],
        scratch_types=[
            pltpu.VMEM((TPW, DIM), jnp.float32),
            pltpu.VMEM((TPW,), jnp.int32),
            pltpu.VMEM((TPW,), jnp.int32),
            pltpu.VMEM((TPW, WV), jnp.float32),
            pltpu.VMEM((TPW, WV), jnp.float32),
            pltpu.SemaphoreType.DMA,
        ],
    )
    def k2(x_hbm, d0_hbm, d1_hbm, w0_hbm, w1_hbm, xs_hbm, ws_hbm,
           xbuf, i0, i1, wb0, wb1, sem):
        wid = lax.axis_index("s") * 2 + lax.axis_index("c")
        base = wid * TPW
        a0 = pltpu.async_copy(x_hbm.at[pl.ds(base, TPW)], xbuf, sem)
        a1 = pltpu.async_copy(d0_hbm.at[pl.ds(base, TPW)], i0, sem)
        a2 = pltpu.async_copy(d1_hbm.at[pl.ds(base, TPW)], i1, sem)
        a3 = pltpu.async_copy(w0_hbm.at[pl.ds(base, TPW)], wb0, sem)
        a4 = pltpu.async_copy(w1_hbm.at[pl.ds(base, TPW)], wb1, sem)
        a0.wait()
        a1.wait()
        a2.wait()
        a3.wait()
        a4.wait()
        c0 = pltpu.async_copy(xbuf, xs_hbm.at[i0], sem)
        c1 = pltpu.async_copy(xbuf, xs_hbm.at[i1], sem)
        c2 = pltpu.async_copy(wb0, ws_hbm.at[i0], sem)
        c3 = pltpu.async_copy(wb1, ws_hbm.at[i1], sem)
        c0.wait()
        c1.wait()
        c2.wait()
        c3.wait()

    return k2(x2, d0, d1, w0w, w1w)


def _ffn_body(texp_ref, tval_ref, gfirst_ref, gslot_ref, gnxt_ref, ghn_ref,
              xs_ref, W1_ref, b1_ref, W2_ref, b2_ref, ws_ref, y_ref,
              w1s, w2s, sem1, sem2):
    i = pl.program_id(0)
    slot = gslot_ref[i]

    def issue(eidx, sl):
        pltpu.make_async_copy(W1_ref.at[eidx], w1s.at[sl], sem1.at[sl]).start()
        pltpu.make_async_copy(W2_ref.at[eidx], w2s.at[sl], sem2.at[sl]).start()

    def drain(eidx, sl):
        pltpu.make_async_copy(W1_ref.at[eidx], w1s.at[sl], sem1.at[sl]).wait()
        pltpu.make_async_copy(W2_ref.at[eidx], w2s.at[sl], sem2.at[sl]).wait()

    @pl.when(i == 0)
    def _prime():
        issue(texp_ref[0], slot)

    @pl.when(gfirst_ref[i] == 1)
    def _group_start():
        @pl.when(ghn_ref[i] == 1)
        def _():
            issue(gnxt_ref[i], 1 - slot)

        drain(texp_ref[i], slot)

    @pl.when(tval_ref[i] > 0)
    def _():
        e = texp_ref[i]
        xt = xs_ref[...]
        h = jnp.dot(xt, w1s[slot], preferred_element_type=jnp.float32)
        h = h + b1_ref[pl.ds(e, 1), :]
        h = 0.5 * h * (1.0 + lax.erf(h * 0.7071067811865476))
        y = jnp.dot(h, w2s[slot], preferred_element_type=jnp.float32)
        y = y + b2_ref[pl.ds(e, 1), :]
        y_ref[...] = y * ws_ref[:, :1]


def _ffn_call(xs, ws, W1, b1, W2, b2, texp, tval, gfirst, gslot, gnxt, ghn):
    grid_spec = pltpu.PrefetchScalarGridSpec(
        num_scalar_prefetch=6,
        grid=(NT,),
        in_specs=[
            pl.BlockSpec((TM, DIM), lambda i, *_: (i, 0)),
            pl.BlockSpec(memory_space=pl.ANY),
            pl.BlockSpec((E, HID), lambda i, *_: (0, 0)),
            pl.BlockSpec(memory_space=pl.ANY),
            pl.BlockSpec((E, DIM), lambda i, *_: (0, 0)),
            pl.BlockSpec((TM, WV), lambda i, *_: (i, 0)),
        ],
        out_specs=pl.BlockSpec((TM, DIM), lambda i, *_: (i, 0)),
        scratch_shapes=[
            pltpu.VMEM((2, DIM, HID), jnp.float32),
            pltpu.VMEM((2, HID, DIM), jnp.float32),
            pltpu.SemaphoreType.DMA((2,)),
            pltpu.SemaphoreType.DMA((2,)),
        ],
    )
    return pl.pallas_call(
        _ffn_body,
        grid_spec=grid_spec,
        out_shape=jax.ShapeDtypeStruct((PMAX, DIM), jnp.float32),
        compiler_params=pltpu.CompilerParams(
            dimension_semantics=("arbitrary",),
        ),
    )(texp, tval, gfirst, gslot, gnxt, ghn, xs, W1, b1, W2, b2, ws)


def _group_meta(te):
    idx = jnp.arange(NT, dtype=jnp.int32)
    first = jnp.concatenate(
        [jnp.ones((1,), jnp.int32),
         (te[1:] != te[:-1]).astype(jnp.int32)])
    gslot = (jnp.cumsum(first) - 1) % 2
    sstart = jnp.where(first == 1, idx, NT + 1)
    suf = jnp.flip(jax.lax.cummin(jnp.flip(sstart)))
    nxtpos = jnp.concatenate([suf[1:], jnp.full((1,), NT + 1, jnp.int32)])
    ghn = (nxtpos <= NT - 1).astype(jnp.int32)
    gnxt = te[jnp.clip(nxtpos, 0, NT - 1)]
    return first, gslot.astype(jnp.int32), gnxt, ghn


def _combine_call(ys, d0, d1):
    mesh = plsc.VectorSubcoreMesh(core_axis_name="c", subcore_axis_name="s")
    half = TPW // 2

    @functools.partial(
        pl.kernel, mesh=mesh,
        out_type=jax.ShapeDtypeStruct((B, S, DIM), jnp.float32),
        scratch_types=[
            pltpu.VMEM((half,), jnp.int32),
            pltpu.VMEM((half,), jnp.int32),
            pltpu.VMEM((half,), jnp.int32),
            pltpu.VMEM((half,), jnp.int32),
            pltpu.VMEM((half, DIM), jnp.float32),
            pltpu.VMEM((half, DIM), jnp.float32),
            pltpu.VMEM((half, DIM), jnp.float32),
            pltpu.VMEM((half, DIM), jnp.float32),
            pltpu.SemaphoreType.DMA,
            pltpu.SemaphoreType.DMA,
            pltpu.SemaphoreType.DMA,
        ],
    )
    def k4(y_hbm, d0_hbm, d1_hbm, out_hbm,
           i0a, i1a, i0b, i1b, ya0, ya1, yb0, yb1, sema, semb, sems):
        wid = lax.axis_index("s") * 2 + lax.axis_index("c")
        base = wid * TPW
        pltpu.sync_copy(d0_hbm.at[pl.ds(base, half)], i0a)
        pltpu.sync_copy(d1_hbm.at[pl.ds(base, half)], i1a)
        ga0 = pltpu.async_copy(y_hbm.at[i0a], ya0, sema)
        ga1 = pltpu.async_copy(y_hbm.at[i1a], ya1, sema)
        pltpu.sync_copy(d0_hbm.at[pl.ds(base + half, half)], i0b)
        pltpu.sync_copy(d1_hbm.at[pl.ds(base + half, half)], i1b)
        gb0 = pltpu.async_copy(y_hbm.at[i0b], yb0, semb)
        gb1 = pltpu.async_copy(y_hbm.at[i1b], yb1, semb)

        def addrows(p0, p1):
            def row(r, carry):
                for cc in range(DIM // 16):
                    sl = pl.ds(cc * 16, 16)
                    p0[r, sl] = p0[r, sl] + p1[r, sl]
                return carry
            lax.fori_loop(0, half, row, 0)

        ga0.wait()
        ga1.wait()
        addrows(ya0, ya1)
        sa = pltpu.async_copy(ya0, out_hbm.at[0, pl.ds(base, half)], sems)
        gb0.wait()
        gb1.wait()
        addrows(yb0, yb1)
        sb = pltpu.async_copy(yb0, out_hbm.at[0, pl.ds(base + half, half)], sems)
        sa.wait()
        sb.wait()

    return k4(ys, d0, d1)


def kernel(x, W1, b1, W2, b2, Wg, bg, temperature):
    x2 = x.reshape(S, DIM)
    bg2 = bg.reshape(1, E)
    temp = jnp.reshape(temperature, (1, 1)).astype(jnp.float32)

    d0c, d1c, w0w, w1w, te, tv = _gate_call(x2, Wg, bg2, temp)
    d0 = d0c.reshape(S)
    d1 = d1c.reshape(S)
    xs, ws = _dispatch_call(x2, d0, d1, w0w, w1w)
    ter = te.reshape(NT)
    gfirst, gslot, gnxt, ghn = _group_meta(ter)
    ys = _ffn_call(xs, ws, W1, b1, W2, b2, ter, tv.reshape(NT),
                   gfirst, gslot, gnxt, ghn)
    return _combine_call(ys, d0, d1)
